# baseline (speedup 1.0000x reference)
import jax
import jax.numpy as jnp
from jax import lax
from jax.experimental import pallas as pl
from jax.experimental.pallas import tpu as pltpu
from jax.experimental.pallas import tpu_sc as plsc

N = 100000
D = 128
G = 256
NC = 2
NS = 16
NW = NC * NS
CHUNK = 128
SC_CHUNKS_PER_TILE = 16
SC_CHUNKS = NW * SC_CHUNKS_PER_TILE
SC_ROWS = SC_CHUNKS * CHUNK
NBUF = 4

TC_BLK = 512
TC_BLK0 = SC_ROWS // TC_BLK
TC_NBLK = -(-(N - SC_ROWS) // TC_BLK)


def _sc_partials(node_feature, seg_ids):
    mesh = plsc.VectorSubcoreMesh(core_axis_name="c", subcore_axis_name="s")

    def body(nf_hbm, ids_hbm, out_hbm, acc_sh,
             r0, r1, r2, r3, x0, x1, x2, x3, stage_v,
             sl0, sl1, sl2, sl3, ss0, ss1, ss2, ss3):
        rows = (r0, r1, r2, r3)
        idxs = (x0, x1, x2, x3)
        slds = (sl0, sl1, sl2, sl3)
        sscs = (ss0, ss1, ss2, ss3)

        c = lax.axis_index("c")
        s = lax.axis_index("s")
        wid = s * NC + c

        zero = jnp.zeros((16,), jnp.float32)
        for i in range(16):
            for j in range(D // 16):
                stage_v[i, pl.ds(j * 16, 16)] = zero
        pltpu.sync_copy(stage_v, acc_sh.at[pl.ds(s * 16, 16)])
        plsc.subcore_barrier()

        def start_load(i, b):
            off = (wid + i * NW) * CHUNK
            pltpu.async_copy(ids_hbm.at[pl.ds(off, CHUNK)], idxs[b], slds[b])
            pltpu.async_copy(nf_hbm.at[pl.ds(off, CHUNK)], rows[b], slds[b])

        def wait_load(b):
            pltpu.make_async_copy(
                ids_hbm.at[pl.ds(0, CHUNK)], idxs[b], slds[b]).wait()
            pltpu.make_async_copy(
                nf_hbm.at[pl.ds(0, CHUNK)], rows[b], slds[b]).wait()

        def start_scatter(b):
            pltpu.async_copy(rows[b], acc_sh.at[idxs[b]], sscs[b], add=True)

        def wait_scatter(b):
            pltpu.make_async_copy(rows[b], acc_sh.at[idxs[b]], sscs[b]).wait()

        start_load(0, 0)

        def k_body(k, carry):
            for b in range(NBUF):
                nb = (b + 1) % NBUF
                if b == NBUF - 1:
                    @pl.when(k < SC_CHUNKS_PER_TILE // NBUF - 1)
                    def _():
                        wait_scatter(nb)
                        start_load(NBUF * (k + 1), nb)
                else:
                    @pl.when(k > 0)
                    def _():
                        wait_scatter(nb)
                    start_load(NBUF * k + b + 1, nb)
                wait_load(b)
                start_scatter(b)
            return carry

        lax.fori_loop(0, SC_CHUNKS_PER_TILE // NBUF, k_body, 0)

        for b in range(NBUF):
            wait_scatter(b)

        plsc.subcore_barrier()

        pltpu.sync_copy(acc_sh.at[pl.ds(s * 16, 16)], stage_v)
        pltpu.sync_copy(stage_v, out_hbm.at[c, pl.ds(s * 16, 16)])

    return pl.kernel(
        body,
        out_type=jax.ShapeDtypeStruct((NC, G, D), jnp.float32),
        mesh=mesh,
        scratch_types=[
            pltpu.VMEM_SHARED((G, D), jnp.float32),
            pltpu.VMEM((CHUNK, D), jnp.float32),
            pltpu.VMEM((CHUNK, D), jnp.float32),
            pltpu.VMEM((CHUNK, D), jnp.float32),
            pltpu.VMEM((CHUNK, D), jnp.float32),
            pltpu.VMEM((CHUNK,), jnp.int32),
            pltpu.VMEM((CHUNK,), jnp.int32),
            pltpu.VMEM((CHUNK,), jnp.int32),
            pltpu.VMEM((CHUNK,), jnp.int32),
            pltpu.VMEM((16, D), jnp.float32),
            pltpu.SemaphoreType.DMA,
            pltpu.SemaphoreType.DMA,
            pltpu.SemaphoreType.DMA,
            pltpu.SemaphoreType.DMA,
            pltpu.SemaphoreType.DMA,
            pltpu.SemaphoreType.DMA,
            pltpu.SemaphoreType.DMA,
            pltpu.SemaphoreType.DMA,
        ],
    )(node_feature, seg_ids)


def _tc_partial(node_feature, seg_ids2d):
    def body(nf_ref, ids_ref, out_ref):
        i = pl.program_id(0)

        @pl.when(i == 0)
        def _():
            out_ref[...] = jnp.zeros((G, D), jnp.float32)

        ids_blk = ids_ref[...]
        seg = lax.broadcasted_iota(jnp.int32, (G, TC_BLK), 0)
        cols = jnp.broadcast_to(ids_blk[None, :], (G, TC_BLK))
        row_g = lax.broadcasted_iota(jnp.int32, (G, TC_BLK), 1) \
            + (TC_BLK0 + i) * TC_BLK
        onehot = jnp.where((seg == cols) & (row_g < N), 1.0, 0.0)
        out_ref[...] += jnp.dot(onehot.astype(jnp.bfloat16),
                                nf_ref[...].astype(jnp.bfloat16),
                                preferred_element_type=jnp.float32)

    return pl.pallas_call(
        body,
        grid=(TC_NBLK,),
        in_specs=[
            pl.BlockSpec((TC_BLK, D), lambda i: (TC_BLK0 + i, 0)),
            pl.BlockSpec((TC_BLK,), lambda i: (TC_BLK0 + i,)),
        ],
        out_specs=pl.BlockSpec((G, D), lambda i: (0, 0)),
        out_shape=jax.ShapeDtypeStruct((G, D), jnp.float32),
    )(node_feature, seg_ids2d)


def _combine(partials, tc_part):
    def body(p_ref, t_ref, o_ref):
        o_ref[...] = p_ref[0] + p_ref[1] + t_ref[...]

    return pl.pallas_call(
        body,
        out_shape=jax.ShapeDtypeStruct((G, D), jnp.float32),
    )(partials, tc_part)


@jax.jit
def kernel(node_feature, segment_ids):
    ids = segment_ids.astype(jnp.int32)
    partials = _sc_partials(node_feature, ids)
    tc_part = _tc_partial(node_feature, ids)
    return _combine(partials, tc_part)


# speedup vs baseline: 1.3582x; 1.3582x over previous
import jax
import jax.numpy as jnp
from jax import lax
from jax.experimental import pallas as pl
from jax.experimental.pallas import tpu as pltpu
from jax.experimental.pallas import tpu_sc as plsc

N = 100000
D = 128
G = 256
NC = 2
NS = 16
NW = NC * NS
CHUNK = 128
SC_CHUNKS_PER_TILE = 20
SC_CHUNKS = NW * SC_CHUNKS_PER_TILE
SC_ROWS = SC_CHUNKS * CHUNK
NBUF = 4

TC_BLK = 512
TC_BLK0 = SC_ROWS // TC_BLK
TC_NBLK = -(-(N - SC_ROWS) // TC_BLK)


def _sc_partials(node_feature, seg_ids):
    mesh = plsc.VectorSubcoreMesh(core_axis_name="c", subcore_axis_name="s")

    def body(nf_hbm, ids_hbm, out_hbm, acc_sh,
             r0, r1, r2, r3, x0, x1, x2, x3, stage_v,
             sl0, sl1, sl2, sl3, ss0, ss1, ss2, ss3):
        rows = (r0, r1, r2, r3)
        idxs = (x0, x1, x2, x3)
        slds = (sl0, sl1, sl2, sl3)
        sscs = (ss0, ss1, ss2, ss3)

        c = lax.axis_index("c")
        s = lax.axis_index("s")
        wid = s * NC + c

        zero = jnp.zeros((16,), jnp.float32)
        for i in range(16):
            for j in range(D // 16):
                stage_v[i, pl.ds(j * 16, 16)] = zero
        pltpu.sync_copy(stage_v, acc_sh.at[pl.ds(s * 16, 16)])
        plsc.subcore_barrier()

        def start_load(i, b):
            off = (wid + i * NW) * CHUNK
            pltpu.async_copy(ids_hbm.at[pl.ds(off, CHUNK)], idxs[b], slds[b])
            pltpu.async_copy(nf_hbm.at[pl.ds(off, CHUNK)], rows[b], slds[b])

        def wait_load(b):
            pltpu.make_async_copy(
                ids_hbm.at[pl.ds(0, CHUNK)], idxs[b], slds[b]).wait()
            pltpu.make_async_copy(
                nf_hbm.at[pl.ds(0, CHUNK)], rows[b], slds[b]).wait()

        def start_scatter(b):
            pltpu.async_copy(rows[b], acc_sh.at[idxs[b]], sscs[b], add=True)

        def wait_scatter(b):
            pltpu.make_async_copy(rows[b], acc_sh.at[idxs[b]], sscs[b]).wait()

        start_load(0, 0)

        def k_body(k, carry):
            for b in range(NBUF):
                nb = (b + 1) % NBUF
                if b == NBUF - 1:
                    @pl.when(k < SC_CHUNKS_PER_TILE // NBUF - 1)
                    def _():
                        wait_scatter(nb)
                        start_load(NBUF * (k + 1), nb)
                else:
                    @pl.when(k > 0)
                    def _():
                        wait_scatter(nb)
                    start_load(NBUF * k + b + 1, nb)
                wait_load(b)
                start_scatter(b)
            return carry

        lax.fori_loop(0, SC_CHUNKS_PER_TILE // NBUF, k_body, 0)

        for b in range(NBUF):
            wait_scatter(b)

        plsc.subcore_barrier()

        pltpu.sync_copy(acc_sh.at[pl.ds(s * 16, 16)], stage_v)
        pltpu.sync_copy(stage_v, out_hbm.at[c, pl.ds(s * 16, 16)])

    return pl.kernel(
        body,
        out_type=jax.ShapeDtypeStruct((NC, G, D), jnp.float32),
        mesh=mesh,
        scratch_types=[
            pltpu.VMEM_SHARED((G, D), jnp.float32),
            pltpu.VMEM((CHUNK, D), jnp.float32),
            pltpu.VMEM((CHUNK, D), jnp.float32),
            pltpu.VMEM((CHUNK, D), jnp.float32),
            pltpu.VMEM((CHUNK, D), jnp.float32),
            pltpu.VMEM((CHUNK,), jnp.int32),
            pltpu.VMEM((CHUNK,), jnp.int32),
            pltpu.VMEM((CHUNK,), jnp.int32),
            pltpu.VMEM((CHUNK,), jnp.int32),
            pltpu.VMEM((16, D), jnp.float32),
            pltpu.SemaphoreType.DMA,
            pltpu.SemaphoreType.DMA,
            pltpu.SemaphoreType.DMA,
            pltpu.SemaphoreType.DMA,
            pltpu.SemaphoreType.DMA,
            pltpu.SemaphoreType.DMA,
            pltpu.SemaphoreType.DMA,
            pltpu.SemaphoreType.DMA,
        ],
    )(node_feature, seg_ids)


def _tc_partial(node_feature, seg_ids2d):
    def body(nf_ref, ids_ref, out_ref):
        i = pl.program_id(0)

        @pl.when(i == 0)
        def _():
            out_ref[...] = jnp.zeros((G, D), jnp.float32)

        ids_blk = ids_ref[...]
        seg = lax.broadcasted_iota(jnp.int32, (G, TC_BLK), 0)
        cols = jnp.broadcast_to(ids_blk[None, :], (G, TC_BLK))
        row_g = lax.broadcasted_iota(jnp.int32, (G, TC_BLK), 1) \
            + (TC_BLK0 + i) * TC_BLK
        onehot = jnp.where((seg == cols) & (row_g < N), 1.0, 0.0)
        out_ref[...] += jnp.dot(onehot, nf_ref[...],
                                preferred_element_type=jnp.float32)

    return pl.pallas_call(
        body,
        grid=(TC_NBLK,),
        in_specs=[
            pl.BlockSpec((TC_BLK, D), lambda i: (TC_BLK0 + i, 0)),
            pl.BlockSpec((TC_BLK,), lambda i: (TC_BLK0 + i,)),
        ],
        out_specs=pl.BlockSpec((G, D), lambda i: (0, 0)),
        out_shape=jax.ShapeDtypeStruct((G, D), jnp.float32),
    )(node_feature, seg_ids2d)


def _combine(partials, tc_part):
    def body(p_ref, t_ref, o_ref):
        o_ref[...] = p_ref[0] + p_ref[1] + t_ref[...]

    return pl.pallas_call(
        body,
        out_shape=jax.ShapeDtypeStruct((G, D), jnp.float32),
    )(partials, tc_part)


@jax.jit
def kernel(node_feature, segment_ids):
    ids = segment_ids.astype(jnp.int32)
    partials = _sc_partials(node_feature, ids)
    tc_part = _tc_partial(node_feature, ids)
    return _combine(partials, tc_part)
